# baseline (speedup 1.0000x reference)
import jax
import jax.numpy as jnp
from jax import lax
from jax.experimental import pallas as pl
from jax.experimental.pallas import tpu as pltpu
from jax.experimental.pallas import tpu_sc as plsc

N = 10000
E = 320000
D_IN = 128
D_HID = 16
D_OUT = 40

NC = 2
NS = 16
L = 16

NW = NC * NS
CW = 128
TBLK = E // CW
NCH = TBLK // NW
XTRA = TBLK - NW * NCH
NP = 10240
RPT = NP // NS

NBUF = 6
NGRP = NCH // NBUF


def _make_seg_agg_body(fuse_relu):
    FW = 2 * L if fuse_relu else L

    def body(f_hbm, adj_hbm, ewm_hbm, b_hbm, out_hbm,
             idx_v, ew_v, xidx_v, xew_v, bias_v, grow, srow, xgrow, xsrow,
             stage_v, acc_sh, gsem, ssem, xsem):
        c = lax.axis_index("c")
        s = lax.axis_index("s")
        wid = c * NS + s

        def _zero(i, _):
            stage_v[i, :] = jnp.zeros((L,), jnp.float32)
            return ()
        lax.fori_loop(0, RPT, _zero, (), unroll=8)
        pltpu.sync_copy(stage_v, acc_sh.at[pl.ds(s * RPT, RPT)])

        base = wid * NCH
        pltpu.sync_copy(adj_hbm.at[pl.ds(base, NCH)], idx_v)
        pltpu.sync_copy(ewm_hbm.at[pl.ds(base, NCH)], ew_v)
        @pl.when(wid < XTRA)
        def _():
            xid = NW * NCH + wid
            pltpu.sync_copy(adj_hbm.at[xid], xidx_v)
            pltpu.sync_copy(ewm_hbm.at[xid], xew_v)
        pltpu.sync_copy(b_hbm, bias_v)
        plsc.subcore_barrier()
        bias = bias_v[...]

        def _gather(b, j):
            return pltpu.make_async_copy(f_hbm.at[idx_v.at[j, 0]], grow[b],
                                         gsem[b])

        def _scatter(b, j):
            pltpu.async_copy(srow[b], acc_sh.at[idx_v.at[j, 1]], ssem[b],
                             add=True)

        def _scatter_wait(b, j):
            pltpu.make_async_copy(srow[b], acc_sh.at[idx_v.at[j, 1]],
                                  ssem[b]).wait()

        def _scale_rows(gbuf, sbuf, w16, q):
            for e in range(L):
                w = jnp.broadcast_to(w16[e], (L,))
                i = q * L + e
                if fuse_relu:
                    row = jnp.maximum(gbuf[i, 0:L] + gbuf[i, L:2 * L], 0.0)
                else:
                    row = gbuf[i, :]
                sbuf[i, :] = row * w

        def _scale(b, j):
            def _sq(q, _):
                _scale_rows(grow[b], srow[b], ew_v[j, q, :], q)
                return ()
            lax.fori_loop(0, CW // L, _sq, ())

        def _xgather():
            return pltpu.make_async_copy(f_hbm.at[xidx_v.at[0]], xgrow, xsem)

        @pl.when(wid < XTRA)
        def _():
            _xgather().start()
        for b in range(NBUF):
            _gather(b, b).start()

        def _group(g, _):
            for b in range(NBUF):
                j = g * NBUF + b
                _gather(b, j).wait()

                @pl.when(g > 0)
                def _():
                    _scatter_wait(b, j - NBUF)

                _scale(b, j)
                _scatter(b, j)

                @pl.when(g < NGRP - 1)
                def _():
                    _gather(b, j + NBUF).start()
            return ()
        lax.fori_loop(0, NGRP, _group, ())

        for b in range(NBUF):
            _scatter_wait(b, (NGRP - 1) * NBUF + b)

        @pl.when(wid < XTRA)
        def _():
            _xgather().wait()
            for q in range(CW // L):
                _scale_rows(xgrow, xsrow, xew_v[q, :], q)
            pltpu.sync_copy(xsrow, acc_sh.at[xidx_v.at[1]], add=True)

        plsc.subcore_barrier()

        pltpu.sync_copy(acc_sh.at[pl.ds(s * RPT, RPT)], stage_v)
        if not fuse_relu:
            hb = bias * 0.5
            def _addb(i, _):
                stage_v[i, :] = stage_v[i, :] + hb
                return ()
            lax.fori_loop(0, RPT, _addb, (), unroll=8)
        if fuse_relu:
            pltpu.sync_copy(stage_v, out_hbm.at[c, pl.ds(s * RPT, RPT)])
        else:
            pltpu.sync_copy(
                stage_v, out_hbm.at[pl.ds(s * RPT, RPT), pl.ds(c * L, L)])

    return body, FW


def _make_seg_agg(fuse_relu):
    body, FW = _make_seg_agg_body(fuse_relu)
    out_shape = ((NC, NP, D_HID) if fuse_relu else (NP, NC * D_HID))
    return pl.kernel(
        body,
        out_type=jax.ShapeDtypeStruct(out_shape, jnp.float32),
        mesh=plsc.VectorSubcoreMesh(core_axis_name="c", subcore_axis_name="s",
                                    num_cores=NC, num_subcores=NS),
        compiler_params=pltpu.CompilerParams(needs_layout_passes=False,
                                             use_tc_tiling_on_sc=False),
        scratch_types=[
            pltpu.VMEM((NCH, 2, CW), jnp.int32),
            pltpu.VMEM((NCH, CW // L, L), jnp.float32),
            pltpu.VMEM((2, CW), jnp.int32),
            pltpu.VMEM((CW // L, L), jnp.float32),
            pltpu.VMEM((L,), jnp.float32),
            [pltpu.VMEM((CW, FW), jnp.float32) for _ in range(NBUF)],
            [pltpu.VMEM((CW, L), jnp.float32) for _ in range(NBUF)],
            pltpu.VMEM((CW, FW), jnp.float32),
            pltpu.VMEM((CW, L), jnp.float32),
            pltpu.VMEM((RPT, L), jnp.float32),
            pltpu.VMEM_SHARED((NP, D_HID), jnp.float32),
            [pltpu.SemaphoreType.DMA for _ in range(NBUF)],
            [pltpu.SemaphoreType.DMA for _ in range(NBUF)],
            pltpu.SemaphoreType.DMA,
        ],
    )


_seg_agg1 = _make_seg_agg(False)
_seg_agg2 = _make_seg_agg(True)


def _mm1_body(x_ref, w_ref, o_ref):
    w = w_ref[...]
    for k in range(8):
        o_ref[:, k * D_HID:(k + 1) * D_HID] = jnp.dot(
            x_ref[:, k, :], w, preferred_element_type=jnp.float32)


def _head_body(p_ref, w_ref, b_ref, o_ref):
    agg = p_ref[0, :N // 8, :] + p_ref[1, :N // 8, :]
    o = jnp.dot(agg, w_ref[...],
                preferred_element_type=jnp.float32) + b_ref[...][None, :]
    for k in range(8):
        blk = o[:, k * D_OUT:(k + 1) * D_OUT]
        blk = blk - jnp.max(blk, axis=1, keepdims=True)
        lse = jnp.log(jnp.sum(jnp.exp(blk), axis=1, keepdims=True))
        o_ref[:, k * D_OUT:(k + 1) * D_OUT] = blk - lse


_mm1 = pl.pallas_call(
    _mm1_body, out_shape=jax.ShapeDtypeStruct((N // 8, 8 * D_HID),
                                              jnp.float32))

_head = pl.pallas_call(
    _head_body, out_shape=jax.ShapeDtypeStruct((N // 8, 8 * D_OUT),
                                               jnp.float32))


def kernel(x, adj_t, edge_weight, W1, b1, W2, b2):
    adj3 = adj_t.astype(jnp.int32).reshape(2, TBLK, CW).transpose(1, 0, 2)
    ew3 = edge_weight.reshape(TBLK, CW // L, L)

    h0 = _mm1(x.reshape(N // 8, 8, D_IN), W1).reshape(N, D_HID)
    p1 = _seg_agg1(h0, adj3, ew3, b1)
    p2 = _seg_agg2(p1, adj3, ew3, b1)
    p2v = p2.reshape(NC, NP // 8, 8 * D_HID)
    W2b = (jnp.eye(8, dtype=jnp.float32)[:, None, :, None]
           * W2[None, :, None, :]).reshape(8 * D_HID, 8 * D_OUT)
    b2b = jnp.tile(b2, 8)
    return _head(p2v, W2b, b2b).reshape(N, D_OUT)


# speedup vs baseline: 1.3211x; 1.3211x over previous
import jax
import jax.numpy as jnp
from jax import lax
from jax.experimental import pallas as pl
from jax.experimental.pallas import tpu as pltpu
from jax.experimental.pallas import tpu_sc as plsc

N = 10000
E = 320000
D_IN = 128
D_HID = 16
D_OUT = 40

NC = 2
NS = 16
L = 16

NW = NC * NS
CW = 128
TBLK = E // CW
NCH = TBLK // NW
XTRA = TBLK - NW * NCH
NP = 10240
RPT = NP // NS

NBUF = 6
NGRP = NCH // NBUF


def _make_seg_agg_body(fuse_relu):
    FW = 2 * L if fuse_relu else L

    def body(f_hbm, adj_hbm, ewm_hbm, b_hbm, out_hbm,
             idx_v, ew_v, xidx_v, xew_v, bias_v, grow, srow, xgrow, xsrow,
             stage_v, acc_sh, gsem, ssem, xsem):
        c = lax.axis_index("c")
        s = lax.axis_index("s")
        wid = c * NS + s

        def _zero(i, _):
            stage_v[i, :] = jnp.zeros((L,), jnp.float32)
            return ()
        lax.fori_loop(0, RPT, _zero, (), unroll=8)
        pltpu.sync_copy(stage_v, acc_sh.at[pl.ds(s * RPT, RPT)])

        base = wid * NCH
        pltpu.sync_copy(adj_hbm.at[pl.ds(base, NCH)], idx_v)
        pltpu.sync_copy(ewm_hbm.at[pl.ds(base, NCH)], ew_v)
        @pl.when(wid < XTRA)
        def _():
            xid = NW * NCH + wid
            pltpu.sync_copy(adj_hbm.at[xid], xidx_v)
            pltpu.sync_copy(ewm_hbm.at[xid], xew_v)
        pltpu.sync_copy(b_hbm, bias_v)
        plsc.subcore_barrier()
        bias = bias_v[...]

        def _gather(b, j):
            return pltpu.make_async_copy(f_hbm.at[idx_v.at[j, 0]], grow[b],
                                         gsem[b])

        def _scatter(b, j):
            pltpu.async_copy(srow[b], acc_sh.at[idx_v.at[j, 1]], ssem[b],
                             add=True)

        def _scatter_wait(b, j):
            pltpu.make_async_copy(srow[b], acc_sh.at[idx_v.at[j, 1]],
                                  ssem[b]).wait()

        def _scale_rows(gbuf, sbuf, w16, q):
            for e in range(L):
                w = jnp.broadcast_to(w16[e], (L,))
                i = q * L + e
                if fuse_relu:
                    row = jnp.maximum(gbuf[i, 0:L] + gbuf[i, L:2 * L], 0.0)
                else:
                    row = gbuf[i, :]
                sbuf[i, :] = row * w

        def _scale(b, j):
            for q in range(CW // L):
                _scale_rows(grow[b], srow[b], ew_v[j, q, :], q)

        def _xgather():
            return pltpu.make_async_copy(f_hbm.at[xidx_v.at[0]], xgrow, xsem)

        @pl.when(wid < XTRA)
        def _():
            _xgather().start()
        for b in range(NBUF):
            _gather(b, b).start()

        def _group(g, _):
            for b in range(NBUF):
                j = g * NBUF + b
                _gather(b, j).wait()

                @pl.when(g > 0)
                def _():
                    _scatter_wait(b, j - NBUF)

                _scale(b, j)
                _scatter(b, j)

                @pl.when(g < NGRP - 1)
                def _():
                    _gather(b, j + NBUF).start()
            return ()
        lax.fori_loop(0, NGRP, _group, ())

        for b in range(NBUF):
            _scatter_wait(b, (NGRP - 1) * NBUF + b)

        @pl.when(wid < XTRA)
        def _():
            _xgather().wait()
            for q in range(CW // L):
                _scale_rows(xgrow, xsrow, xew_v[q, :], q)
            pltpu.sync_copy(xsrow, acc_sh.at[xidx_v.at[1]], add=True)

        plsc.subcore_barrier()

        pltpu.sync_copy(acc_sh.at[pl.ds(s * RPT, RPT)], stage_v)
        if not fuse_relu:
            hb = bias * 0.5
            def _addb(i, _):
                stage_v[i, :] = stage_v[i, :] + hb
                return ()
            lax.fori_loop(0, RPT, _addb, (), unroll=8)
        if fuse_relu:
            pltpu.sync_copy(stage_v, out_hbm.at[c, pl.ds(s * RPT, RPT)])
        else:
            pltpu.sync_copy(
                stage_v, out_hbm.at[pl.ds(s * RPT, RPT), pl.ds(c * L, L)])

    return body, FW


def _make_seg_agg(fuse_relu):
    body, FW = _make_seg_agg_body(fuse_relu)
    out_shape = ((NC, NP, D_HID) if fuse_relu else (NP, NC * D_HID))
    return pl.kernel(
        body,
        out_type=jax.ShapeDtypeStruct(out_shape, jnp.float32),
        mesh=plsc.VectorSubcoreMesh(core_axis_name="c", subcore_axis_name="s",
                                    num_cores=NC, num_subcores=NS),
        compiler_params=pltpu.CompilerParams(needs_layout_passes=False,
                                             use_tc_tiling_on_sc=False),
        scratch_types=[
            pltpu.VMEM((NCH, 2, CW), jnp.int32),
            pltpu.VMEM((NCH, CW // L, L), jnp.float32),
            pltpu.VMEM((2, CW), jnp.int32),
            pltpu.VMEM((CW // L, L), jnp.float32),
            pltpu.VMEM((L,), jnp.float32),
            [pltpu.VMEM((CW, FW), jnp.float32) for _ in range(NBUF)],
            [pltpu.VMEM((CW, L), jnp.float32) for _ in range(NBUF)],
            pltpu.VMEM((CW, FW), jnp.float32),
            pltpu.VMEM((CW, L), jnp.float32),
            pltpu.VMEM((RPT, L), jnp.float32),
            pltpu.VMEM_SHARED((NP, D_HID), jnp.float32),
            [pltpu.SemaphoreType.DMA for _ in range(NBUF)],
            [pltpu.SemaphoreType.DMA for _ in range(NBUF)],
            pltpu.SemaphoreType.DMA,
        ],
    )


_seg_agg1 = _make_seg_agg(False)
_seg_agg2 = _make_seg_agg(True)


def _mm1_body(x_ref, w_ref, o_ref):
    w = w_ref[...]
    for k in range(8):
        o_ref[:, k * D_HID:(k + 1) * D_HID] = jnp.dot(
            x_ref[:, k, :], w, preferred_element_type=jnp.float32)


def _head_body(p_ref, w_ref, b_ref, o_ref):
    agg = p_ref[0, :N // 8, :] + p_ref[1, :N // 8, :]
    o = jnp.dot(agg, w_ref[...],
                preferred_element_type=jnp.float32) + b_ref[...][None, :]
    for k in range(8):
        blk = o[:, k * D_OUT:(k + 1) * D_OUT]
        blk = blk - jnp.max(blk, axis=1, keepdims=True)
        lse = jnp.log(jnp.sum(jnp.exp(blk), axis=1, keepdims=True))
        o_ref[:, k * D_OUT:(k + 1) * D_OUT] = blk - lse


_mm1 = pl.pallas_call(
    _mm1_body, out_shape=jax.ShapeDtypeStruct((N // 8, 8 * D_HID),
                                              jnp.float32))

_head = pl.pallas_call(
    _head_body, out_shape=jax.ShapeDtypeStruct((N // 8, 8 * D_OUT),
                                               jnp.float32))


def kernel(x, adj_t, edge_weight, W1, b1, W2, b2):
    adj3 = adj_t.astype(jnp.int32).reshape(2, TBLK, CW).transpose(1, 0, 2)
    ew3 = edge_weight.reshape(TBLK, CW // L, L)

    h0 = _mm1(x.reshape(N // 8, 8, D_IN), W1).reshape(N, D_HID)
    p1 = _seg_agg1(h0, adj3, ew3, b1)
    p2 = _seg_agg2(p1, adj3, ew3, b1)
    p2v = p2.reshape(NC, NP // 8, 8 * D_HID)
    W2b = (jnp.eye(8, dtype=jnp.float32)[:, None, :, None]
           * W2[None, :, None, :]).reshape(8 * D_HID, 8 * D_OUT)
    b2b = jnp.tile(b2, 8)
    return _head(p2v, W2b, b2b).reshape(N, D_OUT)
